# Initial kernel scaffold; baseline (speedup 1.0000x reference)
import jax
import jax.numpy as jnp
from jax.experimental import pallas as pl


def kernel(x, target):
    raise NotImplementedError("write your pallas kernel here")


# speedup vs baseline: 6.4155x; 6.4155x over previous
import functools
import math

import jax
import jax.numpy as jnp
from jax import lax
from jax.experimental import pallas as pl
from jax.experimental.pallas import tpu as pltpu

_SIZE = 32000
_PAD = 0
_SMOOTH = 0.1
_CONF = 1.0 - _SMOOTH
_EPS = _SMOOTH / (_SIZE - 2)
_ROW_ENT = (_SIZE - 2) * _EPS * math.log(_EPS) + _CONF * math.log(_CONF)

_RB = 512
_CB = 3200


def _tc_body(x_ref, tgt_ref, s_ref, n_ref):
    i = pl.program_id(0)
    j = pl.program_id(1)

    @pl.when((i == 0) & (j == 0))
    def _init():
        s_ref[0, 0] = 0.0
        n_ref[0, 0] = 0.0

    xb = x_ref[...]
    tgt = tgt_ref[...]
    nonpad = tgt != _PAD
    gcol = jax.lax.broadcasted_iota(jnp.int32, xb.shape, 1) + j * _CB
    w = jnp.where(nonpad & (gcol != 0), _EPS, 0.0)
    w = jnp.where(nonpad & (gcol == tgt), _CONF, w)
    s_ref[0, 0] += jnp.sum(w * xb)

    @pl.when(j == 0)
    def _count():
        n_ref[0, 0] += jnp.sum(jnp.where(nonpad, 1.0, 0.0))


@functools.partial(jax.jit, static_argnames=())
def kernel(x, target):
    n_rows = x.shape[0]
    tgt2d = target.astype(jnp.int32).reshape(n_rows, 1)
    grid = (n_rows // _RB, _SIZE // _CB)
    s, n = pl.pallas_call(
        _tc_body,
        grid=grid,
        in_specs=[
            pl.BlockSpec((_RB, _CB), lambda i, j: (i, j)),
            pl.BlockSpec((_RB, 1), lambda i, j: (i, 0)),
        ],
        out_specs=[
            pl.BlockSpec(memory_space=pltpu.MemorySpace.SMEM),
            pl.BlockSpec(memory_space=pltpu.MemorySpace.SMEM),
        ],
        out_shape=[
            jax.ShapeDtypeStruct((1, 1), jnp.float32),
            jax.ShapeDtypeStruct((1, 1), jnp.float32),
        ],
    )(x, tgt2d)
    return _ROW_ENT * n[0, 0] - s[0, 0]
